# baseline (speedup 1.0000x reference)
import functools

import jax
import jax.numpy as jnp
from jax import lax
from jax.experimental import pallas as pl
from jax.experimental.pallas import tpu as pltpu
from jax.experimental.pallas import tpu_sc as plsc

N = 10000
D = 128
E = 320000
NC, NS = 2, 16
NW = NC * NS
SUB = 128
RT = 80
RT0, RT1 = 160, 0
R = NW * RT
E_PAD = R * SUB
ACC_N = 10112
CW = 16
G = 8
ZROWS = ACC_N // NS
BLK = 1000


def _sc_edge_pass():
    mesh = plsc.VectorSubcoreMesh(core_axis_name="c", subcore_axis_name="s")
    out_type = [jax.ShapeDtypeStruct((NC, ACC_N, D), jnp.float32)]
    scratch = [
        pltpu.VMEM((G, SUB), jnp.int32),
        pltpu.VMEM((G, SUB), jnp.int32),
        pltpu.VMEM((SUB, D), jnp.float32),
        pltpu.VMEM((SUB, D), jnp.float32),
        pltpu.VMEM_SHARED((ACC_N, D), jnp.float32),
        pltpu.SemaphoreType.DMA,
        pltpu.SemaphoreType.DMA,
    ]

    def body(p, src, dst, zeros, acc_out,
             src_v, dst_v, rows_a, rows_b, acc_sh, sem_a, sem_b):
        c = lax.axis_index("c")
        s = lax.axis_index("s")
        pltpu.sync_copy(zeros.at[pl.ds(s * ZROWS, ZROWS)],
                        acc_sh.at[pl.ds(s * ZROWS, ZROWS)])
        plsc.subcore_barrier()
        bufs = [(rows_a, sem_a), (rows_b, sem_b)]

        def run(base, nrows):
            def step(g, carry):
                pltpu.sync_copy(src.at[pl.ds(base + g * G, G)], src_v)
                pltpu.sync_copy(dst.at[pl.ds(base + g * G, G)], dst_v)
                cps = [pltpu.async_copy(p.at[src_v.at[0]],
                                        bufs[0][0], bufs[0][1])]
                for j in range(G):
                    buf, _ = bufs[j % 2]
                    if j + 1 < G:
                        nbuf, nsem = bufs[(j + 1) % 2]
                        cps.append(
                            pltpu.async_copy(p.at[src_v.at[j + 1]], nbuf, nsem))
                    cps[j].wait()
                    pltpu.sync_copy(buf, acc_sh.at[dst_v.at[j]], add=True)
                return carry

            lax.fori_loop(0, nrows // G, step, 0)

        @pl.when(c == 0)
        def _():
            run(s * RT0, RT0)

        @pl.when(c == 1)
        def _():
            run(NS * RT0 + s * RT1, RT1)

        plsc.subcore_barrier()
        pltpu.sync_copy(acc_sh.at[pl.ds(s * ZROWS, ZROWS)],
                        acc_out.at[c, pl.ds(s * ZROWS, ZROWS)])

    return pl.kernel(body, out_type=out_type, mesh=mesh,
                     scratch_types=scratch)


def _sc_deg_pass():
    mesh = plsc.VectorSubcoreMesh(core_axis_name="c", subcore_axis_name="s")
    out_type = [jax.ShapeDtypeStruct((NC, ACC_N, D), jnp.float32)]
    scratch = [
        pltpu.VMEM((G, SUB), jnp.int32),
        pltpu.VMEM((SUB, D), jnp.float32),
        pltpu.VMEM_SHARED((ACC_N, D), jnp.float32),
    ]

    def body(dst, ones, zeros, cnt_out, dst_v, ones_v, acc_sh):
        c = lax.axis_index("c")
        s = lax.axis_index("s")
        wid = s * NC + c
        base = wid * RT
        pltpu.sync_copy(ones, ones_v)
        pltpu.sync_copy(zeros.at[pl.ds(s * ZROWS, ZROWS)],
                        acc_sh.at[pl.ds(s * ZROWS, ZROWS)])
        plsc.subcore_barrier()

        def step(g, carry):
            pltpu.sync_copy(dst.at[pl.ds(base + g * G, G)], dst_v)
            for j in range(G):
                pltpu.sync_copy(ones_v, acc_sh.at[dst_v.at[j]], add=True)
            return carry

        lax.fori_loop(0, RT // G, step, 0)
        plsc.subcore_barrier()
        pltpu.sync_copy(acc_sh.at[pl.ds(s * ZROWS, ZROWS)],
                        cnt_out.at[c, pl.ds(s * ZROWS, ZROWS)])

    return pl.kernel(body, out_type=out_type, mesh=mesh,
                     scratch_types=scratch)


_edge_pass = _sc_edge_pass()
_deg_pass = _sc_deg_pass()


def _mm_pre(x, ws, wn, b):
    def body(x_r, ws_r, wn_r, b_r, s_r, p_r):
        xb = x_r[...]
        s_r[...] = jnp.dot(xb, ws_r[...], preferred_element_type=jnp.float32) + b_r[...]
        p_r[...] = jnp.dot(xb, wn_r[...], preferred_element_type=jnp.float32)

    return pl.pallas_call(
        body,
        grid=(N // BLK,),
        in_specs=[pl.BlockSpec((BLK, D), lambda i: (i, 0)),
                  pl.BlockSpec((D, D), lambda i: (0, 0)),
                  pl.BlockSpec((D, D), lambda i: (0, 0)),
                  pl.BlockSpec((1, D), lambda i: (0, 0))],
        out_specs=[pl.BlockSpec((BLK, D), lambda i: (i, 0)),
                   pl.BlockSpec((BLK, D), lambda i: (i, 0))],
        out_shape=[jax.ShapeDtypeStruct((N, D), jnp.float32)] * 2,
    )(x, ws, wn, b.reshape(1, D))


def _mm_mid(s1, p1, a0, a1, c0, c1, ws, wn, b):
    def body(s1_r, p1_r, a0_r, a1_r, c0_r, c1_r, ws_r, wn_r, b_r, s_r, p_r):
        deg = 1.0 + c0_r[:, 0:1] + c1_r[:, 0:1]
        agg = p1_r[...] + a0_r[...] + a1_r[...]
        h = jnp.maximum(s1_r[...] + agg / deg, 0.0)
        s_r[...] = jnp.dot(h, ws_r[...], preferred_element_type=jnp.float32) + b_r[...]
        p_r[...] = jnp.dot(h, wn_r[...], preferred_element_type=jnp.float32)

    blk = lambda w: pl.BlockSpec((BLK, w), lambda i: (i, 0))
    return pl.pallas_call(
        body,
        grid=(N // BLK,),
        in_specs=[blk(D), blk(D), blk(D), blk(D), blk(D), blk(D),
                  pl.BlockSpec((D, D), lambda i: (0, 0)),
                  pl.BlockSpec((D, D), lambda i: (0, 0)),
                  pl.BlockSpec((1, D), lambda i: (0, 0))],
        out_specs=[blk(D), blk(D)],
        out_shape=[jax.ShapeDtypeStruct((N, D), jnp.float32)] * 2,
    )(s1, p1, a0, a1, c0, c1, ws, wn, b.reshape(1, D))


def _mm_post(s2, p2, a0, a1, c0, c1):
    def body(s2_r, p2_r, a0_r, a1_r, c0_r, c1_r, o_r):
        deg = 1.0 + c0_r[:, 0:1] + c1_r[:, 0:1]
        o_r[...] = s2_r[...] + (p2_r[...] + a0_r[...] + a1_r[...]) / deg

    blk = lambda w: pl.BlockSpec((BLK, w), lambda i: (i, 0))
    return pl.pallas_call(
        body,
        grid=(N // BLK,),
        in_specs=[blk(D), blk(D), blk(D), blk(D), blk(D), blk(D)],
        out_specs=blk(D),
        out_shape=jax.ShapeDtypeStruct((N, D), jnp.float32),
    )(s2, p2, a0, a1, c0, c1)


def kernel(x, edge_index, W_self1, W_neigh1, b1, W_self2, W_neigh2, b2):
    src = edge_index[0].astype(jnp.int32)
    dst = edge_index[1].astype(jnp.int32)
    pad = E_PAD - E
    src_p = jnp.concatenate([src, jnp.zeros((pad,), jnp.int32)]).reshape(R, SUB)
    dst_p = jnp.concatenate([dst, jnp.full((pad,), N, jnp.int32)]).reshape(R, SUB)
    zeros = jnp.zeros((ACC_N, D), jnp.float32)
    ones_row = jnp.ones((SUB, D), jnp.float32)

    s1, p1 = _mm_pre(x, W_self1, W_neigh1, b1)
    (cnt,) = _deg_pass(dst_p, ones_row, zeros)
    (acc1,) = _edge_pass(p1, src_p, dst_p, zeros)
    s2, p2 = _mm_mid(s1, p1, acc1[0], acc1[1], cnt[0], cnt[1],
                     W_self2, W_neigh2, b2)
    (acc2,) = _edge_pass(p2, src_p, dst_p, zeros)
    return _mm_post(s2, p2, acc2[0], acc2[1], cnt[0], cnt[1])


# speedup vs baseline: 1.0702x; 1.0702x over previous
import functools

import jax
import jax.numpy as jnp
from jax import lax
from jax.experimental import pallas as pl
from jax.experimental.pallas import tpu as pltpu
from jax.experimental.pallas import tpu_sc as plsc

N = 10000
D = 128
E = 320000
NC, NS = 2, 16
NW = NC * NS
SUB = 128
RT = 80
RT0, RT1 = 80, 80
HN = 10240
HC = HN // NS
R = NW * RT
E_PAD = R * SUB
ACC_N = 10112
CW = 16
G = 8
ZROWS = ACC_N // NS
BLK = 1000


def _sc_edge_pass():
    mesh = plsc.VectorSubcoreMesh(core_axis_name="c", subcore_axis_name="s")
    out_type = [jax.ShapeDtypeStruct((NC, ACC_N, D), jnp.float32)]
    scratch = [
        pltpu.VMEM((G, SUB), jnp.int32),
        pltpu.VMEM((G, SUB), jnp.int32),
        pltpu.VMEM((SUB, D), jnp.float32),
        pltpu.VMEM((SUB, D), jnp.float32),
        pltpu.VMEM_SHARED((ACC_N, D), jnp.float32),
        pltpu.SemaphoreType.DMA,
        pltpu.SemaphoreType.DMA,
    ]

    def body(p, src, dst, zeros, acc_out,
             src_v, dst_v, rows_a, rows_b, acc_sh, sem_a, sem_b):
        c = lax.axis_index("c")
        s = lax.axis_index("s")
        pltpu.sync_copy(zeros.at[pl.ds(s * ZROWS, ZROWS)],
                        acc_sh.at[pl.ds(s * ZROWS, ZROWS)])
        plsc.subcore_barrier()
        bufs = [(rows_a, sem_a), (rows_b, sem_b)]

        def run(base, nrows):
            def step(g, carry):
                pltpu.sync_copy(src.at[pl.ds(base + g * G, G)], src_v)
                pltpu.sync_copy(dst.at[pl.ds(base + g * G, G)], dst_v)
                cps = [pltpu.async_copy(p.at[src_v.at[0]],
                                        bufs[0][0], bufs[0][1])]
                for j in range(G):
                    buf, _ = bufs[j % 2]
                    if j + 1 < G:
                        nbuf, nsem = bufs[(j + 1) % 2]
                        cps.append(
                            pltpu.async_copy(p.at[src_v.at[j + 1]], nbuf, nsem))
                    cps[j].wait()
                    pltpu.sync_copy(buf, acc_sh.at[dst_v.at[j]], add=True)
                return carry

            lax.fori_loop(0, nrows // G, step, 0)

        @pl.when(c == 0)
        def _():
            run(s * RT0, RT0)

        @pl.when(c == 1)
        def _():
            run(NS * RT0 + s * RT1, RT1)

        plsc.subcore_barrier()
        pltpu.sync_copy(acc_sh.at[pl.ds(s * ZROWS, ZROWS)],
                        acc_out.at[c, pl.ds(s * ZROWS, ZROWS)])

    return pl.kernel(body, out_type=out_type, mesh=mesh,
                     scratch_types=scratch)


def _sc_deg_pass():
    mesh = plsc.VectorSubcoreMesh(core_axis_name="c", subcore_axis_name="s")
    out_type = [jax.ShapeDtypeStruct((NC, ACC_N, D), jnp.float32)]
    scratch = [
        pltpu.VMEM((G, SUB), jnp.int32),
        pltpu.VMEM((SUB, D), jnp.float32),
        pltpu.VMEM_SHARED((ACC_N, D), jnp.float32),
    ]

    def body(dst, zeros, ones, acc_out, dst_v, ones_v, acc_sh):
        c = lax.axis_index("c")
        s = lax.axis_index("s")
        pltpu.sync_copy(zeros.at[pl.ds(s * ZROWS, ZROWS)],
                        acc_sh.at[pl.ds(s * ZROWS, ZROWS)])
        pltpu.sync_copy(ones, ones_v)
        plsc.subcore_barrier()

        def run(base, nrows):
            def step(g, carry):
                pltpu.sync_copy(dst.at[pl.ds(base + g * G, G)], dst_v)
                for j in range(G):
                    pltpu.sync_copy(ones_v, acc_sh.at[dst_v.at[j]], add=True)
                return carry

            lax.fori_loop(0, nrows // G, step, 0)

        @pl.when(c == 0)
        def _():
            run(s * RT0, RT0)

        @pl.when(c == 1)
        def _():
            run(NS * RT0 + s * RT1, RT1)

        plsc.subcore_barrier()
        pltpu.sync_copy(acc_sh.at[pl.ds(s * ZROWS, ZROWS)],
                        acc_out.at[c, pl.ds(s * ZROWS, ZROWS)])

    return pl.kernel(body, out_type=out_type, mesh=mesh,
                     scratch_types=scratch)


_edge_pass = _sc_edge_pass()
_deg_pass = _sc_deg_pass()


def _mm_pre(x, ws, wn, b):
    def body(x_r, ws_r, wn_r, b_r, s_r, p_r):
        xb = x_r[...]
        s_r[...] = jnp.dot(xb, ws_r[...], preferred_element_type=jnp.float32) + b_r[...]
        p_r[...] = jnp.dot(xb, wn_r[...], preferred_element_type=jnp.float32)

    return pl.pallas_call(
        body,
        grid=(N // BLK,),
        in_specs=[pl.BlockSpec((BLK, D), lambda i: (i, 0)),
                  pl.BlockSpec((D, D), lambda i: (0, 0)),
                  pl.BlockSpec((D, D), lambda i: (0, 0)),
                  pl.BlockSpec((1, D), lambda i: (0, 0))],
        out_specs=[pl.BlockSpec((BLK, D), lambda i: (i, 0)),
                   pl.BlockSpec((BLK, D), lambda i: (i, 0))],
        out_shape=[jax.ShapeDtypeStruct((N, D), jnp.float32)] * 2,
    )(x, ws, wn, b.reshape(1, D))


def _mm_mid(s1, p1, a0, a1, c0, c1, ws, wn, b):
    def body(s1_r, p1_r, a0_r, a1_r, c0_r, c1_r, ws_r, wn_r, b_r, s_r, p_r):
        deg = 1.0 + c0_r[...] + c1_r[...]
        agg = p1_r[...] + a0_r[...] + a1_r[...]
        h = jnp.maximum(s1_r[...] + agg / deg, 0.0)
        s_r[...] = jnp.dot(h, ws_r[...], preferred_element_type=jnp.float32) + b_r[...]
        p_r[...] = jnp.dot(h, wn_r[...], preferred_element_type=jnp.float32)

    blk = lambda w: pl.BlockSpec((BLK, w), lambda i: (i, 0))
    return pl.pallas_call(
        body,
        grid=(N // BLK,),
        in_specs=[blk(D), blk(D), blk(D), blk(D), blk(1), blk(1),
                  pl.BlockSpec((D, D), lambda i: (0, 0)),
                  pl.BlockSpec((D, D), lambda i: (0, 0)),
                  pl.BlockSpec((1, D), lambda i: (0, 0))],
        out_specs=[blk(D), blk(D)],
        out_shape=[jax.ShapeDtypeStruct((N, D), jnp.float32)] * 2,
    )(s1, p1, a0, a1, c0, c1, ws, wn, b.reshape(1, D))


def _mm_post(s2, p2, a0, a1, c0, c1):
    def body(s2_r, p2_r, a0_r, a1_r, c0_r, c1_r, o_r):
        deg = 1.0 + c0_r[...] + c1_r[...]
        o_r[...] = s2_r[...] + (p2_r[...] + a0_r[...] + a1_r[...]) / deg

    blk = lambda w: pl.BlockSpec((BLK, w), lambda i: (i, 0))
    return pl.pallas_call(
        body,
        grid=(N // BLK,),
        in_specs=[blk(D), blk(D), blk(D), blk(D), blk(1), blk(1)],
        out_specs=blk(D),
        out_shape=jax.ShapeDtypeStruct((N, D), jnp.float32),
    )(s2, p2, a0, a1, c0, c1)


def kernel(x, edge_index, W_self1, W_neigh1, b1, W_self2, W_neigh2, b2):
    src = edge_index[0].astype(jnp.int32)
    dst = edge_index[1].astype(jnp.int32)
    pad = E_PAD - E
    src_p = jnp.concatenate([src, jnp.zeros((pad,), jnp.int32)]).reshape(R, SUB)
    dst_p = jnp.concatenate([dst, jnp.full((pad,), N, jnp.int32)]).reshape(R, SUB)
    zeros = jnp.zeros((ACC_N, D), jnp.float32)

    ones_pay = jnp.ones((SUB, D), jnp.float32)

    s1, p1 = _mm_pre(x, W_self1, W_neigh1, b1)
    (accd,) = _deg_pass(dst_p, zeros, ones_pay)
    c0, c1 = accd[0, :N, :1], accd[1, :N, :1]
    (acc1,) = _edge_pass(p1, src_p, dst_p, zeros)
    s2, p2 = _mm_mid(s1, p1, acc1[0], acc1[1], c0, c1,
                     W_self2, W_neigh2, b2)
    (acc2,) = _edge_pass(p2, src_p, dst_p, zeros)
    return _mm_post(s2, p2, acc2[0], acc2[1], c0, c1)


# speedup vs baseline: 1.1061x; 1.0335x over previous
import functools

import jax
import jax.numpy as jnp
from jax import lax
from jax.experimental import pallas as pl
from jax.experimental.pallas import tpu as pltpu
from jax.experimental.pallas import tpu_sc as plsc

N = 10000
D = 128
E = 320000
NC, NS = 2, 16
NW = NC * NS
SUB = 128
RT = 80
RT0, RT1 = 80, 80
HN = 10240
HC = HN // NS
R = NW * RT
E_PAD = R * SUB
ACC_N = 10112
CW = 16
G = 8
ZROWS = ACC_N // NS
BLK = 1000


def _sc_edge_pass():
    mesh = plsc.VectorSubcoreMesh(core_axis_name="c", subcore_axis_name="s")
    out_type = [jax.ShapeDtypeStruct((NC, ACC_N, D), jnp.float32)]
    scratch = [
        pltpu.VMEM((G, SUB), jnp.int32),
        pltpu.VMEM((G, SUB), jnp.int32),
        pltpu.VMEM((G, SUB), jnp.int32),
        pltpu.VMEM((G, SUB), jnp.int32),
        pltpu.VMEM((SUB, D), jnp.float32),
        pltpu.VMEM((SUB, D), jnp.float32),
        pltpu.VMEM_SHARED((ACC_N, D), jnp.float32),
        pltpu.SemaphoreType.DMA,
        pltpu.SemaphoreType.DMA,
        pltpu.SemaphoreType.DMA,
        pltpu.SemaphoreType.DMA,
    ]

    def body(p, src, dst, zeros, acc_out,
             src_v0, dst_v0, src_v1, dst_v1, rows_a, rows_b, acc_sh,
             sem_a, sem_b, sem_is, sem_id):
        c = lax.axis_index("c")
        s = lax.axis_index("s")
        pltpu.sync_copy(zeros.at[pl.ds(s * ZROWS, ZROWS)],
                        acc_sh.at[pl.ds(s * ZROWS, ZROWS)])
        plsc.subcore_barrier()
        bufs = [(rows_a, sem_a), (rows_b, sem_b)]
        ibufs = [(src_v0, dst_v0), (src_v1, dst_v1)]
        base = c * (NS * RT) + s * RT
        nchunks = RT // G
        pltpu.sync_copy(src.at[pl.ds(base, G)], src_v0)
        pltpu.sync_copy(dst.at[pl.ds(base, G)], dst_v0)
        hand = [None, None]
        hand[0] = pltpu.async_copy(p.at[src_v0.at[0]], rows_a, sem_a)
        idx_pend = None
        for k in range(nchunks):
            cs, cd = ibufs[k % 2]
            if k + 1 < nchunks:
                ns_, nd_ = ibufs[(k + 1) % 2]
                idx_pend = (
                    pltpu.async_copy(src.at[pl.ds(base + (k + 1) * G, G)],
                                     ns_, sem_is),
                    pltpu.async_copy(dst.at[pl.ds(base + (k + 1) * G, G)],
                                     nd_, sem_id),
                )
            for j in range(G):
                r = k * G + j
                buf, _ = bufs[r % 2]
                nbuf, nsem = bufs[(r + 1) % 2]
                if j + 1 < G:
                    hand[(r + 1) % 2] = pltpu.async_copy(
                        p.at[cs.at[j + 1]], nbuf, nsem)
                elif k + 1 < nchunks:
                    idx_pend[0].wait()
                    idx_pend[1].wait()
                    hand[(r + 1) % 2] = pltpu.async_copy(
                        p.at[ns_.at[0]], nbuf, nsem)
                hand[r % 2].wait()
                pltpu.sync_copy(buf, acc_sh.at[cd.at[j]], add=True)

        plsc.subcore_barrier()
        pltpu.sync_copy(acc_sh.at[pl.ds(s * ZROWS, ZROWS)],
                        acc_out.at[c, pl.ds(s * ZROWS, ZROWS)])

    return pl.kernel(body, out_type=out_type, mesh=mesh,
                     scratch_types=scratch)


def _sc_deg_pass():
    mesh = plsc.VectorSubcoreMesh(core_axis_name="c", subcore_axis_name="s")
    out_type = [jax.ShapeDtypeStruct((NC, ACC_N, D), jnp.float32)]
    scratch = [
        pltpu.VMEM((G, SUB), jnp.int32),
        pltpu.VMEM((SUB, D), jnp.float32),
        pltpu.VMEM_SHARED((ACC_N, D), jnp.float32),
    ]

    def body(dst, zeros, ones, acc_out, dst_v, ones_v, acc_sh):
        c = lax.axis_index("c")
        s = lax.axis_index("s")
        pltpu.sync_copy(zeros.at[pl.ds(s * ZROWS, ZROWS)],
                        acc_sh.at[pl.ds(s * ZROWS, ZROWS)])
        pltpu.sync_copy(ones, ones_v)
        plsc.subcore_barrier()

        def run(base, nrows):
            def step(g, carry):
                pltpu.sync_copy(dst.at[pl.ds(base + g * G, G)], dst_v)
                for j in range(G):
                    pltpu.sync_copy(ones_v, acc_sh.at[dst_v.at[j]], add=True)
                return carry

            lax.fori_loop(0, nrows // G, step, 0)

        @pl.when(c == 0)
        def _():
            run(s * RT0, RT0)

        @pl.when(c == 1)
        def _():
            run(NS * RT0 + s * RT1, RT1)

        plsc.subcore_barrier()
        pltpu.sync_copy(acc_sh.at[pl.ds(s * ZROWS, ZROWS)],
                        acc_out.at[c, pl.ds(s * ZROWS, ZROWS)])

    return pl.kernel(body, out_type=out_type, mesh=mesh,
                     scratch_types=scratch)


_edge_pass = _sc_edge_pass()
_deg_pass = _sc_deg_pass()


def _mm_pre(x, ws, wn, b):
    def body(x_r, ws_r, wn_r, b_r, s_r, p_r):
        xb = x_r[...]
        s_r[...] = jnp.dot(xb, ws_r[...], preferred_element_type=jnp.float32) + b_r[...]
        p_r[...] = jnp.dot(xb, wn_r[...], preferred_element_type=jnp.float32)

    return pl.pallas_call(
        body,
        grid=(N // BLK,),
        in_specs=[pl.BlockSpec((BLK, D), lambda i: (i, 0)),
                  pl.BlockSpec((D, D), lambda i: (0, 0)),
                  pl.BlockSpec((D, D), lambda i: (0, 0)),
                  pl.BlockSpec((1, D), lambda i: (0, 0))],
        out_specs=[pl.BlockSpec((BLK, D), lambda i: (i, 0)),
                   pl.BlockSpec((BLK, D), lambda i: (i, 0))],
        out_shape=[jax.ShapeDtypeStruct((N, D), jnp.float32)] * 2,
    )(x, ws, wn, b.reshape(1, D))


def _mm_mid(s1, p1, a0, a1, c0, c1, ws, wn, b):
    def body(s1_r, p1_r, a0_r, a1_r, c0_r, c1_r, ws_r, wn_r, b_r, s_r, p_r):
        deg = 1.0 + c0_r[...] + c1_r[...]
        agg = p1_r[...] + a0_r[...] + a1_r[...]
        h = jnp.maximum(s1_r[...] + agg / deg, 0.0)
        s_r[...] = jnp.dot(h, ws_r[...], preferred_element_type=jnp.float32) + b_r[...]
        p_r[...] = jnp.dot(h, wn_r[...], preferred_element_type=jnp.float32)

    blk = lambda w: pl.BlockSpec((BLK, w), lambda i: (i, 0))
    return pl.pallas_call(
        body,
        grid=(N // BLK,),
        in_specs=[blk(D), blk(D), blk(D), blk(D), blk(1), blk(1),
                  pl.BlockSpec((D, D), lambda i: (0, 0)),
                  pl.BlockSpec((D, D), lambda i: (0, 0)),
                  pl.BlockSpec((1, D), lambda i: (0, 0))],
        out_specs=[blk(D), blk(D)],
        out_shape=[jax.ShapeDtypeStruct((N, D), jnp.float32)] * 2,
    )(s1, p1, a0, a1, c0, c1, ws, wn, b.reshape(1, D))


def _mm_post(s2, p2, a0, a1, c0, c1):
    def body(s2_r, p2_r, a0_r, a1_r, c0_r, c1_r, o_r):
        deg = 1.0 + c0_r[...] + c1_r[...]
        o_r[...] = s2_r[...] + (p2_r[...] + a0_r[...] + a1_r[...]) / deg

    blk = lambda w: pl.BlockSpec((BLK, w), lambda i: (i, 0))
    return pl.pallas_call(
        body,
        grid=(N // BLK,),
        in_specs=[blk(D), blk(D), blk(D), blk(D), blk(1), blk(1)],
        out_specs=blk(D),
        out_shape=jax.ShapeDtypeStruct((N, D), jnp.float32),
    )(s2, p2, a0, a1, c0, c1)


def kernel(x, edge_index, W_self1, W_neigh1, b1, W_self2, W_neigh2, b2):
    src = edge_index[0].astype(jnp.int32)
    dst = edge_index[1].astype(jnp.int32)
    pad = E_PAD - E
    src_p = jnp.concatenate([src, jnp.zeros((pad,), jnp.int32)]).reshape(R, SUB)
    dst_p = jnp.concatenate([dst, jnp.full((pad,), N, jnp.int32)]).reshape(R, SUB)
    zeros = jnp.zeros((ACC_N, D), jnp.float32)

    ones_pay = jnp.ones((SUB, D), jnp.float32)

    s1, p1 = _mm_pre(x, W_self1, W_neigh1, b1)
    (accd,) = _deg_pass(dst_p, zeros, ones_pay)
    c0, c1 = accd[0, :N, :1], accd[1, :N, :1]
    (acc1,) = _edge_pass(p1, src_p, dst_p, zeros)
    s2, p2 = _mm_mid(s1, p1, acc1[0], acc1[1], c0, c1,
                     W_self2, W_neigh2, b2)
    (acc2,) = _edge_pass(p2, src_p, dst_p, zeros)
    return _mm_post(s2, p2, acc2[0], acc2[1], c0, c1)
